# Initial kernel scaffold; baseline (speedup 1.0000x reference)
import jax
import jax.numpy as jnp
from jax.experimental import pallas as pl


def kernel(aa_types, seq_rep, aa_table, W_seq, b_seq, W_tok, b_tok, g_seq, be_seq, g_tok, be_tok):
    raise NotImplementedError("write your pallas kernel here")


# speedup vs baseline: 2.3648x; 2.3648x over previous
import functools

import jax
import jax.numpy as jnp
from jax import lax
from jax.experimental import pallas as pl
from jax.experimental.pallas import tpu as pltpu
from jax.experimental.pallas import tpu_sc as plsc

_EPS = 1e-5
_NC = 2
_NS = 16
_NW = _NC * _NS
_CHUNK = 128


def _sc_gather(table, idx3d, latent):
    chunks_per_w = idx3d.shape[1]
    n_idx = _NW * chunks_per_w * _CHUNK
    mesh = plsc.VectorSubcoreMesh(core_axis_name="c", subcore_axis_name="s")

    @functools.partial(
        pl.kernel,
        mesh=mesh,
        out_type=jax.ShapeDtypeStruct((n_idx, latent), jnp.float32),
        scratch_types=[
            pltpu.VMEM((chunks_per_w, _CHUNK), jnp.int32),
            pltpu.VMEM((_CHUNK, latent), jnp.float32),
            pltpu.SemaphoreType.DMA,
        ],
        compiler_params=pltpu.CompilerParams(use_tc_tiling_on_sc=False),
    )
    def k(table_hbm, idx_hbm, out_hbm, idx_v, rows_v, sem):
        wid = lax.axis_index("s") * _NC + lax.axis_index("c")
        crow0 = wid * chunks_per_w
        pltpu.sync_copy(idx_hbm.at[wid], idx_v)

        def body(j, carry):
            pltpu.async_copy(table_hbm.at[idx_v.at[j]], rows_v, sem).wait()
            pltpu.sync_copy(rows_v, out_hbm.at[pl.ds((crow0 + j) * _CHUNK, _CHUNK)])
            return carry

        lax.fori_loop(0, chunks_per_w, body, 0)

    return k(table, idx3d)


def _tc_dense(seq_rep, aa3, Wst, bs, Wtt, bt, gs, bes, gt, bet):
    B, L, D = seq_rep.shape
    latent = aa3.shape[-1]
    bB = 64
    grid = (B // bB,)

    def body(seq_ref, aa_ref, wst_ref, bs_ref, wtt_ref, bt_ref,
             gs_ref, bes_ref, gt_ref, bet_ref, out_ref):
        x = seq_ref[...]
        m = jnp.mean(x, axis=-1, keepdims=True)
        xc = x - m
        v = jnp.mean(xc * xc, axis=-1, keepdims=True)
        xn = xc * lax.rsqrt(v + _EPS)
        xn = xn * gt_ref[...].reshape(1, 1, D) + bet_ref[...].reshape(1, 1, D)
        tok = jnp.dot(xn.reshape(bB * L, D), wtt_ref[...],
                      preferred_element_type=jnp.float32)
        tok = tok.reshape(bB, L, latent) + bt_ref[...].reshape(1, 1, latent)
        sm = jnp.mean(x, axis=1)
        m2 = jnp.mean(sm, axis=-1, keepdims=True)
        sc = sm - m2
        v2 = jnp.mean(sc * sc, axis=-1, keepdims=True)
        sn = sc * lax.rsqrt(v2 + _EPS) * gs_ref[...] + bes_ref[...]
        se = jnp.dot(sn, wst_ref[...], preferred_element_type=jnp.float32)
        se = se + bs_ref[...]
        out_ref[...] = tok + se.reshape(bB, 1, latent) + aa_ref[...]

    return pl.pallas_call(
        body,
        grid=grid,
        in_specs=[
            pl.BlockSpec((bB, L, D), lambda i: (i, 0, 0)),
            pl.BlockSpec((bB, L, latent), lambda i: (i, 0, 0)),
            pl.BlockSpec((D, latent), lambda i: (0, 0)),
            pl.BlockSpec((1, latent), lambda i: (0, 0)),
            pl.BlockSpec((D, latent), lambda i: (0, 0)),
            pl.BlockSpec((1, latent), lambda i: (0, 0)),
            pl.BlockSpec((1, D), lambda i: (0, 0)),
            pl.BlockSpec((1, D), lambda i: (0, 0)),
            pl.BlockSpec((1, D), lambda i: (0, 0)),
            pl.BlockSpec((1, D), lambda i: (0, 0)),
        ],
        out_specs=pl.BlockSpec((bB, L, latent), lambda i: (i, 0, 0)),
        out_shape=jax.ShapeDtypeStruct((B, L, latent), jnp.float32),
    )(seq_rep, aa3, Wst, bs, Wtt, bt, gs, bes, gt, bet)


def kernel(aa_types, seq_rep, aa_table, W_seq, b_seq, W_tok, b_tok,
           g_seq, be_seq, g_tok, be_tok):
    B, L, D = seq_rep.shape
    latent = aa_table.shape[-1]
    idx3d = aa_types.astype(jnp.int32).reshape(_NW, -1, _CHUNK)
    aa_flat = _sc_gather(aa_table, idx3d, latent)
    aa3 = aa_flat.reshape(B, L, latent)
    return _tc_dense(
        seq_rep, aa3,
        W_seq.T, b_seq.reshape(1, -1),
        W_tok.T, b_tok.reshape(1, -1),
        g_seq.reshape(1, -1), be_seq.reshape(1, -1),
        g_tok.reshape(1, -1), be_tok.reshape(1, -1),
    )


# speedup vs baseline: 2.4972x; 1.0560x over previous
import functools

import jax
import jax.numpy as jnp
from jax import lax
from jax.experimental import pallas as pl
from jax.experimental.pallas import tpu as pltpu
from jax.experimental.pallas import tpu_sc as plsc

_EPS = 1e-5
_NC = 2
_NS = 16
_NW = _NC * _NS
_CHUNK = 128


def _sc_gather(table, idx3d, latent):
    chunks_per_w = idx3d.shape[1]
    n_idx = _NW * chunks_per_w * _CHUNK
    mesh = plsc.VectorSubcoreMesh(core_axis_name="c", subcore_axis_name="s")

    @functools.partial(
        pl.kernel,
        mesh=mesh,
        out_type=jax.ShapeDtypeStruct((n_idx, latent), jnp.float32),
        scratch_types=[
            pltpu.VMEM((chunks_per_w, _CHUNK), jnp.int32),
            pltpu.VMEM((_CHUNK, latent), jnp.float32),
            pltpu.SemaphoreType.DMA,
        ],
        compiler_params=pltpu.CompilerParams(use_tc_tiling_on_sc=False),
    )
    def k(table_hbm, idx_hbm, out_hbm, idx_v, rows_v, sem):
        wid = lax.axis_index("s") * _NC + lax.axis_index("c")
        crow0 = wid * chunks_per_w
        pltpu.sync_copy(idx_hbm.at[wid], idx_v)

        def body(j, carry):
            pltpu.async_copy(table_hbm.at[idx_v.at[j]], rows_v, sem).wait()
            pltpu.sync_copy(rows_v, out_hbm.at[pl.ds((crow0 + j) * _CHUNK,
                                                     _CHUNK)])
            return carry

        lax.fori_loop(0, chunks_per_w, body, 0)

    return k(table, idx3d)


def _tc_dense(seq_rep, aa2w, Wst, bs, Wtt, bt, gs, bes, gt, bet):
    B, L, D = seq_rep.shape
    latent = aa2w.shape[-1] // 2
    H = L // 2
    bB = 32
    grid = (B // bB,)

    def body(seq_ref, aa_ref, wst_ref, bs_ref, wtt_ref, bt_ref,
             gs_ref, bes_ref, gt_ref, bet_ref, out_ref):
        x = seq_ref[...]
        m = jnp.mean(x, axis=-1, keepdims=True)
        xc = x - m
        v = jnp.mean(xc * xc, axis=-1, keepdims=True)
        xn = xc * lax.rsqrt(v + _EPS)
        xn = xn * gt_ref[...].reshape(1, 1, D) + bet_ref[...].reshape(1, 1, D)
        xp = xn.reshape(bB, H, 2, D)
        w_tok = wtt_ref[...]
        b_tok = bt_ref[...].reshape(1, 1, latent)
        tokE = jnp.dot(xp[:, :, 0, :].reshape(bB * H, D), w_tok,
                       preferred_element_type=jnp.float32).reshape(bB, H, latent)
        tokO = jnp.dot(xp[:, :, 1, :].reshape(bB * H, D), w_tok,
                       preferred_element_type=jnp.float32).reshape(bB, H, latent)
        sm = jnp.mean(x, axis=1)
        m2 = jnp.mean(sm, axis=-1, keepdims=True)
        sc = sm - m2
        v2 = jnp.mean(sc * sc, axis=-1, keepdims=True)
        sn = sc * lax.rsqrt(v2 + _EPS) * gs_ref[...] + bes_ref[...]
        se = jnp.dot(sn, wst_ref[...], preferred_element_type=jnp.float32)
        se = (se + bs_ref[...]).reshape(bB, 1, latent)
        aa = aa_ref[...].reshape(bB, H, 2 * latent)
        addE = tokE + b_tok + se + aa[:, :, :latent]
        addO = tokO + b_tok + se + aa[:, :, latent:]
        out_ref[...] = jnp.concatenate([addE, addO], axis=-1)

    out2w = pl.pallas_call(
        body,
        grid=grid,
        in_specs=[
            pl.BlockSpec((bB, L, D), lambda i: (i, 0, 0)),
            pl.BlockSpec((bB * H, 2 * latent), lambda i: (i, 0)),
            pl.BlockSpec((D, latent), lambda i: (0, 0)),
            pl.BlockSpec((1, latent), lambda i: (0, 0)),
            pl.BlockSpec((D, latent), lambda i: (0, 0)),
            pl.BlockSpec((1, latent), lambda i: (0, 0)),
            pl.BlockSpec((1, D), lambda i: (0, 0)),
            pl.BlockSpec((1, D), lambda i: (0, 0)),
            pl.BlockSpec((1, D), lambda i: (0, 0)),
            pl.BlockSpec((1, D), lambda i: (0, 0)),
        ],
        out_specs=pl.BlockSpec((bB, H, 2 * latent), lambda i: (i, 0, 0)),
        out_shape=jax.ShapeDtypeStruct((B, H, 2 * latent), jnp.float32),
    )(seq_rep, aa2w, Wst, bs, Wtt, bt, gs, bes, gt, bet)
    return out2w


def kernel(aa_types, seq_rep, aa_table, W_seq, b_seq, W_tok, b_tok,
           g_seq, be_seq, g_tok, be_tok):
    B, L, D = seq_rep.shape
    latent = aa_table.shape[-1]
    idx3d = aa_types.astype(jnp.int32).reshape(_NW, -1, _CHUNK)
    aa_flat = _sc_gather(aa_table, idx3d, latent)
    aa2w = aa_flat.reshape(B * L // 2, 2 * latent)
    out2w = _tc_dense(
        seq_rep, aa2w,
        W_seq.T, b_seq.reshape(1, -1),
        W_tok.T, b_tok.reshape(1, -1),
        g_seq.reshape(1, -1), be_seq.reshape(1, -1),
        g_tok.reshape(1, -1), be_tok.reshape(1, -1),
    )
    return out2w.reshape(B, L, latent)
